# baseline (speedup 1.0000x reference)
import functools
import numpy as np
import jax
import jax.numpy as jnp
from jax import lax
from jax.experimental import pallas as pl
from jax.experimental.pallas import tpu as pltpu
from jax.experimental.pallas import tpu_sc as plsc

H = W = 48
C = 96
HW = H * W
HALF = 6
PAD = HALF * W
MAXC = 16
GR = 8
GS = GR * W
GB = (GR + 2 * HALF) * W
NG = H // GR
SBC = 1024
APR = 32
KWIN = (2 * HALF + 1) ** 2
NW = 32
PPW = HW // NW
NVK = 11
HI = lax.Precision.HIGHEST
NEG = -1e9


def _resize_mat(in_size, out_size):
    inv = in_size / out_size
    ks = max(inv, 1.0)
    sample_f = (np.arange(out_size) + 0.5) * inv - 0.5
    t = np.abs(sample_f[None, :] - np.arange(in_size)[:, None]) / ks
    w = np.maximum(0.0, 1.0 - t)
    w = w / np.sum(w, axis=0, keepdims=True)
    return w.T.astype(np.float32)


_D_MAT = _resize_mat(H, H // 2)
_U_MAT = _resize_mat(H // 2, H)

_M_W = (_U_MAT.astype(np.float64) @ _D_MAT.astype(np.float64))
_DBAND = 3


def _shift_coefs():
    cf = np.zeros((C * W, 2 * _DBAND + 1), np.float32)
    for wp in range(W):
        for j, d in enumerate(range(-_DBAND, _DBAND + 1)):
            if 0 <= wp + d < W:
                cf[wp::W, j] = np.float32(_M_W[wp, wp + d])
    return cf


_CF = _shift_coefs()


def _tc1_body(xpad2_ref, xt2_ref, xt_ref, d_ref, u_ref, cf_ref,
              s_ref, kct_ref, pf_ref):
    f32 = jnp.float32
    dmat = d_ref[...]
    umat = u_ref[...]

    xt = xt_ref[...]
    p1 = lax.dot_general(xt, dmat, (((1,), (1,)), ((), ())),
                         preferred_element_type=f32, precision=HI)
    p2 = lax.dot_general(p1, umat, (((1,), (1,)), ((), ())),
                         preferred_element_type=f32, precision=HI)
    p2p = jnp.pad(p2, ((_DBAND, _DBAND), (0, 0)))
    cf = cf_ref[...]
    up = cf[:, 0:1] * lax.slice(p2p, (0, 0), (C * W, H))
    for j in range(1, 2 * _DBAND + 1):
        up = up + cf[:, j:j + 1] * lax.slice(p2p, (j, 0), (j + C * W, H))
    dabs = jnp.abs(xt - up)
    for rows in (2304, 1152, 576, 288, 144):
        dabs = lax.slice(dabs, (0, 0), (rows, H)) + \
            lax.slice(dabs, (rows, 0), (2 * rows, H))
    dft = (lax.slice(dabs, (0, 0), (W, H)) +
           lax.slice(dabs, (W, 0), (2 * W, H)) +
           lax.slice(dabs, (2 * W, 0), (3 * W, H)))
    mn = jnp.min(dft)
    mx = jnp.max(dft)
    kct_ref[...] = 1.0 + jnp.round((dft - mn) / (mx - mn + 1e-8) * (MAXC - 1))

    xpad = xpad2_ref[pl.ds(APR, HW + 2 * PAD), :]
    nrm = jnp.sqrt(jnp.sum(xpad * xpad, axis=1, keepdims=True))
    pf_ref[...] = xpad / jnp.maximum(nrm, 1e-12)
    xt2 = xt2_ref[...]
    nrmt = jnp.maximum(jnp.sqrt(jnp.sum(xt2 * xt2, axis=0, keepdims=True)),
                       1e-12)

    ss = lax.broadcasted_iota(jnp.int32, (GS, GB), 0)
    jj = lax.broadcasted_iota(jnp.int32, (GS, GB), 1)
    gi = ss // W
    px = ss - gi * W
    bj = jj // W
    cx = jj - bj * W
    colmask = jnp.abs(cx - px) <= HALF
    dyrel = bj - gi
    winmask = colmask & (dyrel >= 0) & (dyrel <= 2 * HALF)
    apron = jnp.full((GS, APR), NEG, f32)

    for g in range(NG):
        y0 = g * GR
        rows_pf = pf_ref[pl.ds((y0 + HALF) * W, GS), :]
        band_u = xpad2_ref[pl.ds(y0 * W + APR, GB), :]
        sraw = lax.dot_general(rows_pf, band_u, (((1,), (1,)), ((), ())),
                               preferred_element_type=f32)
        bnrm = lax.slice(nrmt, (0, y0 * W), (1, y0 * W + GB))
        s = sraw / bnrm
        rowvalid = (y0 + bj - HALF >= 0) & (y0 + bj - HALF < H)
        sm = jnp.where(winmask & rowvalid, s, NEG)
        s_ref[pl.ds(y0 * W, GS), :] = jnp.concatenate(
            [apron, sm, apron], axis=1)


def _sc_body(s_hbm, out_hbm, s_v, o_v, sem):
    i32 = jnp.int32
    wid = lax.axis_index("s") * 2 + lax.axis_index("c")
    pltpu.async_copy(s_hbm.at[pl.ds(wid * PPW * SBC, PPW * SBC)], s_v,
                     sem).wait()
    lanes = lax.iota(i32, 16)
    offs, valids = [], []
    for v in range(NVK):
        k = lanes + 16 * v
        offs.append((k // 13) * W + (k % 13))
        valids.append(k < KWIN)

    def pix_body(p, carry):
        n = wid * PPW + p
        y = n // W
        i = y % GR
        x = n - y * W
        rb = p * SBC
        cb = rb + i * W + x + (APR - HALF)
        top = jnp.full((16,), NEG, jnp.float32)
        for v in range(NVK):
            idx = jnp.where(valids[v], cb + offs[v], rb)
            vals = plsc.load_gather(s_v, [idx])
            vs = lax.sort(vals)
            top = lax.sort(jnp.maximum(top, lax.rev(vs, (0,))))
        plsc.store_scatter(o_v, [p * 16 + lanes], top)
        return carry

    lax.fori_loop(0, PPW, pix_body, 0)
    pltpu.sync_copy(o_v, out_hbm.at[pl.ds(wid * PPW * 16, PPW * 16)])


def _sc_top16(s_flat):
    mesh = plsc.VectorSubcoreMesh(core_axis_name="c", subcore_axis_name="s",
                                  num_cores=2, num_subcores=16)
    return pl.kernel(
        _sc_body,
        out_type=jax.ShapeDtypeStruct((HW * 16,), jnp.float32),
        mesh=mesh,
        compiler_params=pltpu.CompilerParams(needs_layout_passes=False),
        scratch_types=[
            pltpu.VMEM((PPW * SBC,), jnp.float32),
            pltpu.VMEM((PPW * 16,), jnp.float32),
            pltpu.SemaphoreType.DMA,
        ],
    )(s_flat)


def _tc2_body(xpad2_ref, s_ref, t16_ref, kct_ref, lnw_ref, lnb_ref,
              w1_ref, b1_ref, w2_ref, b2_ref, out_ref):
    f32 = jnp.float32
    kct8 = jnp.concatenate([kct_ref[...]] * GR, axis=0)
    hh48 = lax.broadcasted_iota(jnp.int32, (GS, H), 1)
    gi48 = lax.broadcasted_iota(jnp.int32, (GS, H), 0) // W
    kk16 = lax.broadcasted_iota(jnp.int32, (GS, 16), 1)

    lnw = lnw_ref[...]
    lnb = lnb_ref[...]
    w1 = w1_ref[...]
    b1 = b1_ref[...]
    w2 = w2_ref[...]
    b2 = b2_ref[...]

    for g in range(NG):
        y0 = g * GR
        kc = jnp.sum(jnp.where(hh48 == (y0 + gi48), kct8, 0.0),
                     axis=1, keepdims=True)
        t16 = t16_ref[pl.ds(y0 * W, GS), :]
        kci = lax.convert_element_type(kc, jnp.int32)
        thr = jnp.sum(jnp.where(kk16 == (16 - kci), t16, 0.0),
                      axis=1, keepdims=True)

        sm = s_ref[pl.ds(y0 * W, GS), :]
        w_un = jnp.where(sm >= thr, jnp.exp(sm), 0.0)
        z = jnp.sum(w_un, axis=1, keepdims=True)
        wn = w_un / z

        band_u = xpad2_ref[pl.ds(y0 * W, SBC), :]
        agg = lax.dot_general(wn, band_u, (((1,), (0,)), ((), ())),
                              preferred_element_type=f32, precision=HI)

        xrow_u = xpad2_ref[pl.ds((y0 + HALF) * W + APR, GS), :]
        mu = jnp.mean(xrow_u, axis=1, keepdims=True)
        ctr = xrow_u - mu
        var = jnp.mean(ctr * ctr, axis=1, keepdims=True)
        xln = ctr / jnp.sqrt(var + 1e-5) * lnw + lnb

        enh = agg + xln
        hmid = lax.dot_general(enh, w1, (((1,), (1,)), ((), ())),
                               preferred_element_type=f32) + b1
        hmid = jnp.maximum(hmid, 0.0)
        ffn = lax.dot_general(hmid, w2, (((1,), (1,)), ((), ())),
                              preferred_element_type=f32) + b2
        out_ref[pl.ds(y0 * W, GS), :] = enh + ffn


def kernel(x, ln_w, ln_b, w1, b1, w2, b2):
    xf = x[0].transpose(1, 2, 0).reshape(HW, C)
    xpad2 = jnp.pad(xf, ((PAD + APR, PAD + APR + 32), (0, 0)))
    xpad = jnp.pad(xf, ((PAD, PAD), (0, 0)))
    xt2 = xpad.T
    xt = x[0].transpose(0, 2, 1).reshape(C * W, H)

    s_flat, kct = pl.pallas_call(
        _tc1_body,
        out_shape=(jax.ShapeDtypeStruct((HW, SBC), jnp.float32),
                   jax.ShapeDtypeStruct((W, H), jnp.float32)),
        scratch_shapes=[pltpu.VMEM((HW + 2 * PAD, C), jnp.float32)],
    )(xpad2, xt2, xt, jnp.asarray(_D_MAT), jnp.asarray(_U_MAT),
      jnp.asarray(_CF))

    t16 = _sc_top16(s_flat.reshape(HW * SBC)).reshape(HW, 16)

    out = pl.pallas_call(
        _tc2_body,
        out_shape=jax.ShapeDtypeStruct((HW, C), jnp.float32),
    )(xpad2, s_flat, t16, kct, ln_w.reshape(1, C), ln_b.reshape(1, C),
      w1, b1.reshape(1, 2 * C), w2, b2.reshape(1, C))

    return out.reshape(H, W, C).transpose(2, 0, 1)[None]


# speedup vs baseline: 1.1980x; 1.1980x over previous
import functools
import numpy as np
import jax
import jax.numpy as jnp
from jax import lax
from jax.experimental import pallas as pl
from jax.experimental.pallas import tpu as pltpu
from jax.experimental.pallas import tpu_sc as plsc

H = W = 48
C = 96
HW = H * W
HALF = 6
PAD = HALF * W
MAXC = 16
GR = 8
GS = GR * W
GB = (GR + 2 * HALF) * W
NG = H // GR
SBC = 1024
APR = 32
KWIN = (2 * HALF + 1) ** 2
NW = 32
PPW = HW // NW
NVK = 11
HI = lax.Precision.HIGHEST
NEG = -1e9


def _resize_mat(in_size, out_size):
    inv = in_size / out_size
    ks = max(inv, 1.0)
    sample_f = (np.arange(out_size) + 0.5) * inv - 0.5
    t = np.abs(sample_f[None, :] - np.arange(in_size)[:, None]) / ks
    w = np.maximum(0.0, 1.0 - t)
    w = w / np.sum(w, axis=0, keepdims=True)
    return w.T.astype(np.float32)


_D_MAT = _resize_mat(H, H // 2)
_U_MAT = _resize_mat(H // 2, H)

_M_W = (_U_MAT.astype(np.float64) @ _D_MAT.astype(np.float64))
_DBAND = 3


def _shift_coefs():
    cf = np.zeros((C * W, 2 * _DBAND + 1), np.float32)
    for wp in range(W):
        for j, d in enumerate(range(-_DBAND, _DBAND + 1)):
            if 0 <= wp + d < W:
                cf[wp::W, j] = np.float32(_M_W[wp, wp + d])
    return cf


_CF = _shift_coefs()


def _tc1_body(xpad2_ref, xt2_ref, xt_ref, d_ref, u_ref, cf_ref,
              s_ref, kct_ref, pf_ref):
    f32 = jnp.float32
    dmat = d_ref[...]
    umat = u_ref[...]

    xt = xt_ref[...]
    p1 = lax.dot_general(xt, dmat, (((1,), (1,)), ((), ())),
                         preferred_element_type=f32, precision=HI)
    p2 = lax.dot_general(p1, umat, (((1,), (1,)), ((), ())),
                         preferred_element_type=f32, precision=HI)
    p2p = jnp.pad(p2, ((_DBAND, _DBAND), (0, 0)))
    cf = cf_ref[...]
    up = cf[:, 0:1] * lax.slice(p2p, (0, 0), (C * W, H))
    for j in range(1, 2 * _DBAND + 1):
        up = up + cf[:, j:j + 1] * lax.slice(p2p, (j, 0), (j + C * W, H))
    dabs = jnp.abs(xt - up)
    for rows in (2304, 1152, 576, 288, 144):
        dabs = lax.slice(dabs, (0, 0), (rows, H)) + \
            lax.slice(dabs, (rows, 0), (2 * rows, H))
    dft = (lax.slice(dabs, (0, 0), (W, H)) +
           lax.slice(dabs, (W, 0), (2 * W, H)) +
           lax.slice(dabs, (2 * W, 0), (3 * W, H)))
    mn = jnp.min(dft)
    mx = jnp.max(dft)
    kct_ref[...] = 1.0 + jnp.round((dft - mn) / (mx - mn + 1e-8) * (MAXC - 1))

    xpad = xpad2_ref[pl.ds(APR, HW + 2 * PAD), :]
    nrm = jnp.sqrt(jnp.sum(xpad * xpad, axis=1, keepdims=True))
    pf_ref[...] = xpad / jnp.maximum(nrm, 1e-12)
    xt2 = xt2_ref[...]
    nrmt = jnp.maximum(jnp.sqrt(jnp.sum(xt2 * xt2, axis=0, keepdims=True)),
                       1e-12)

    ss = lax.broadcasted_iota(jnp.int32, (GS, GB), 0)
    jj = lax.broadcasted_iota(jnp.int32, (GS, GB), 1)
    gi = ss // W
    px = ss - gi * W
    bj = jj // W
    cx = jj - bj * W
    colmask = jnp.abs(cx - px) <= HALF
    dyrel = bj - gi
    winmask = colmask & (dyrel >= 0) & (dyrel <= 2 * HALF)
    apron = jnp.full((GS, APR), NEG, f32)

    for g in range(NG):
        y0 = g * GR
        rows_pf = pf_ref[pl.ds((y0 + HALF) * W, GS), :]
        band_u = xpad2_ref[pl.ds(y0 * W + APR, GB), :]
        sraw = lax.dot_general(rows_pf, band_u, (((1,), (1,)), ((), ())),
                               preferred_element_type=f32)
        bnrm = lax.slice(nrmt, (0, y0 * W), (1, y0 * W + GB))
        s = sraw / bnrm
        rowvalid = (y0 + bj - HALF >= 0) & (y0 + bj - HALF < H)
        sm = jnp.where(winmask & rowvalid, s, NEG)
        s_ref[pl.ds(y0 * W, GS), :] = jnp.concatenate(
            [apron, sm, apron], axis=1)


def _sc_body(s_hbm, out_hbm, s_v, o_v, sem):
    i32 = jnp.int32
    wid = lax.axis_index("s") * 2 + lax.axis_index("c")
    pltpu.async_copy(s_hbm.at[pl.ds(wid * PPW, PPW)], s_v, sem).wait()
    lanes = lax.iota(i32, 16)
    offs, valids = [], []
    for v in range(NVK):
        k = lanes + 16 * v
        offs.append((k // 13) * W + (k % 13))
        valids.append(k < KWIN)

    def pix_body(p, carry):
        n = wid * PPW + p
        y = n // W
        i = y % GR
        x = n - y * W
        cb = i * W + x + (APR - HALF)
        prow = jnp.full((16,), p, i32)
        top = jnp.full((16,), NEG, jnp.float32)
        for v in range(NVK):
            idx = jnp.where(valids[v], cb + offs[v], 0)
            vals = plsc.load_gather(s_v, [prow, idx])
            vs = lax.sort(vals)
            top = lax.sort(jnp.maximum(top, lax.rev(vs, (0,))))
        plsc.store_scatter(o_v, [prow, lanes], top)
        return carry

    lax.fori_loop(0, PPW, pix_body, 0)
    pltpu.sync_copy(o_v, out_hbm.at[pl.ds(wid * PPW, PPW)])


def _sc_top16(s_flat):
    mesh = plsc.VectorSubcoreMesh(core_axis_name="c", subcore_axis_name="s",
                                  num_cores=2, num_subcores=16)
    return pl.kernel(
        _sc_body,
        out_type=jax.ShapeDtypeStruct((HW, 16), jnp.float32),
        mesh=mesh,
        compiler_params=pltpu.CompilerParams(needs_layout_passes=False),
        scratch_types=[
            pltpu.VMEM((PPW, SBC), jnp.float32),
            pltpu.VMEM((PPW, 16), jnp.float32),
            pltpu.SemaphoreType.DMA,
        ],
    )(s_flat)


def _tc2_body(xpad2_ref, s_ref, t16_ref, kct_ref, lnw_ref, lnb_ref,
              w1_ref, b1_ref, w2_ref, b2_ref, out_ref):
    f32 = jnp.float32
    kct8 = jnp.concatenate([kct_ref[...]] * GR, axis=0)
    hh48 = lax.broadcasted_iota(jnp.int32, (GS, H), 1)
    gi48 = lax.broadcasted_iota(jnp.int32, (GS, H), 0) // W
    kk16 = lax.broadcasted_iota(jnp.int32, (GS, 16), 1)

    lnw = lnw_ref[...]
    lnb = lnb_ref[...]
    w1 = w1_ref[...]
    b1 = b1_ref[...]
    w2 = w2_ref[...]
    b2 = b2_ref[...]

    for g in range(NG):
        y0 = g * GR
        kc = jnp.sum(jnp.where(hh48 == (y0 + gi48), kct8, 0.0),
                     axis=1, keepdims=True)
        t16 = t16_ref[pl.ds(y0 * W, GS), :]
        kci = lax.convert_element_type(kc, jnp.int32)
        thr = jnp.sum(jnp.where(kk16 == (16 - kci), t16, 0.0),
                      axis=1, keepdims=True)

        sm = s_ref[pl.ds(y0 * W, GS), :]
        w_un = jnp.where(sm >= thr, jnp.exp(sm), 0.0)
        z = jnp.sum(w_un, axis=1, keepdims=True)
        wn = w_un / z

        band_u = xpad2_ref[pl.ds(y0 * W, SBC), :]
        agg = lax.dot_general(wn, band_u, (((1,), (0,)), ((), ())),
                              preferred_element_type=f32, precision=HI)

        xrow_u = xpad2_ref[pl.ds((y0 + HALF) * W + APR, GS), :]
        mu = jnp.mean(xrow_u, axis=1, keepdims=True)
        ctr = xrow_u - mu
        var = jnp.mean(ctr * ctr, axis=1, keepdims=True)
        xln = ctr / jnp.sqrt(var + 1e-5) * lnw + lnb

        enh = agg + xln
        hmid = lax.dot_general(enh, w1, (((1,), (1,)), ((), ())),
                               preferred_element_type=f32) + b1
        hmid = jnp.maximum(hmid, 0.0)
        ffn = lax.dot_general(hmid, w2, (((1,), (1,)), ((), ())),
                              preferred_element_type=f32) + b2
        out_ref[pl.ds(y0 * W, GS), :] = enh + ffn


def kernel(x, ln_w, ln_b, w1, b1, w2, b2):
    xf = x[0].transpose(1, 2, 0).reshape(HW, C)
    xpad2 = jnp.pad(xf, ((PAD + APR, PAD + APR + 32), (0, 0)))
    xpad = jnp.pad(xf, ((PAD, PAD), (0, 0)))
    xt2 = xpad.T
    xt = x[0].transpose(0, 2, 1).reshape(C * W, H)

    s_flat, kct = pl.pallas_call(
        _tc1_body,
        out_shape=(jax.ShapeDtypeStruct((HW, SBC), jnp.float32),
                   jax.ShapeDtypeStruct((W, H), jnp.float32)),
        scratch_shapes=[pltpu.VMEM((HW + 2 * PAD, C), jnp.float32)],
    )(xpad2, xt2, xt, jnp.asarray(_D_MAT), jnp.asarray(_U_MAT),
      jnp.asarray(_CF))

    t16 = _sc_top16(s_flat)

    out = pl.pallas_call(
        _tc2_body,
        out_shape=jax.ShapeDtypeStruct((HW, C), jnp.float32),
    )(xpad2, s_flat, t16, kct, ln_w.reshape(1, C), ln_b.reshape(1, C),
      w1, b1.reshape(1, 2 * C), w2, b2.reshape(1, C))

    return out.reshape(H, W, C).transpose(2, 0, 1)[None]
